# baseline (speedup 1.0000x reference)
import functools

import jax
import jax.numpy as jnp
from jax import lax
from jax.experimental import pallas as pl
from jax.experimental.pallas import tpu as pltpu
from jax.experimental.pallas import tpu_sc as plsc

NE = 8
TOPK = 2
S = 4096
A = TOPK * S
H = 1024
BLK = 256
FB = 512
NB = 40
RP = NB * BLK
NW = 32
NT = 16
GCH = 80
CCH = 32
SCH = 128
APT = A // NT
ZPT = RP // NT


def _scan_cols(a):
    n = a.shape[1]
    s = 1
    while s < n:
        shifted = jnp.concatenate(
            [jnp.zeros((a.shape[0], s), a.dtype),
             lax.slice(a, (0, 0), (a.shape[0], n - s))], axis=1)
        a = a + shifted
        s *= 2
    return a


def _router_body(x_ref, gw_ref, logits_ref, pos_ref, w_ref, meta_ref):
    x = x_ref[...]
    gw = gw_ref[...]
    logits_ref[...] = lax.dot_general(x, gw, (((1,), (1,)), ((), ())),
                                      preferred_element_type=jnp.float32)
    lt = lax.dot_general(gw, x, (((1,), (1,)), ((), ())),
                         preferred_element_type=jnp.float32)
    m = jnp.max(lt, axis=0, keepdims=True)
    e = jnp.exp(lt - m)
    p = e / jnp.sum(e, axis=0, keepdims=True)
    iota_e = lax.broadcasted_iota(jnp.int32, p.shape, 0)
    v0 = jnp.max(p, axis=0, keepdims=True)
    i0 = jnp.min(jnp.where(p == v0, iota_e, NE), axis=0, keepdims=True)
    p2 = jnp.where(iota_e == i0, -1.0, p)
    v1 = jnp.max(p2, axis=0, keepdims=True)
    i1 = jnp.min(jnp.where(p2 == v1, iota_e, NE), axis=0, keepdims=True)
    s = v0 + v1

    oh0 = (iota_e == i0).astype(jnp.int32)
    oh1 = (iota_e == i1).astype(jnp.int32)
    c0 = _scan_cols(oh0)
    c1 = _scan_cols(oh1)
    rank0 = jnp.sum((c0 - oh0) * oh0, axis=0, keepdims=True)
    rank1 = jnp.sum((c1 - oh1) * oh1, axis=0, keepdims=True)
    counts0 = lax.slice(c0, (0, S - 1), (NE, S))
    counts1 = lax.slice(c1, (0, S - 1), (NE, S))
    counts = counts0 + counts1
    padded = ((counts + BLK - 1) // BLK) * BLK
    tril = (lax.broadcasted_iota(jnp.int32, (NE, NE), 1)
            <= lax.broadcasted_iota(jnp.int32, (NE, NE), 0)).astype(jnp.float32)
    off_end = lax.dot_general(tril, padded.astype(jnp.float32),
                              (((1,), (0,)), ((), ())),
                              preferred_element_type=jnp.float32
                              ).astype(jnp.int32)
    off = off_end - padded
    pos0 = jnp.sum(oh0 * off, axis=0, keepdims=True) + rank0
    pos1 = jnp.sum(oh1 * (off + counts0), axis=0, keepdims=True) + rank1
    pos_row = jnp.concatenate([pos0, pos1], axis=1)
    pos_ref[...] = jnp.concatenate(
        [pos_row, jnp.zeros((NE - 1, A), jnp.int32)], axis=0)
    w_row = jnp.concatenate([v0 / s, v1 / s], axis=1)
    w_ref[...] = jnp.concatenate(
        [w_row, jnp.zeros((NE - 1, A), jnp.float32)], axis=0)

    n_used = lax.slice(off_end, (NE - 1, 0), (NE, 1)) // BLK
    bcmp = lax.broadcasted_iota(jnp.int32, (NE, NB), 1) * BLK
    be_raw = jnp.minimum(
        jnp.sum((bcmp >= off_end).astype(jnp.int32), axis=0, keepdims=True),
        NE - 1)
    iota_e8 = lax.broadcasted_iota(jnp.int32, (NE, 1), 0)
    be_last = jnp.max(jnp.where(counts > 0, iota_e8, 0), axis=0, keepdims=True)
    bcol = lax.broadcasted_iota(jnp.int32, (1, NB), 1)
    be = jnp.where(bcol >= n_used, be_last, be_raw)
    meta_ref[...] = jnp.concatenate(
        [be, jnp.broadcast_to(n_used, (1, NE))], axis=1)


def _gemm_body(m_ref, xs_ref, rw_ref, w1_ref, w3_ref, w2_ref, ys_ref):
    b = pl.program_id(0)
    f = pl.program_id(1)

    @pl.when(b < m_ref[0, NB])
    def _():
        xb = xs_ref[...]
        g = lax.dot_general(xb, w1_ref[0], (((1,), (1,)), ((), ())),
                            preferred_element_type=jnp.float32)
        u = lax.dot_general(xb, w3_ref[0], (((1,), (1,)), ((), ())),
                            preferred_element_type=jnp.float32)
        h = g * lax.logistic(g) * u * rw_ref[0]
        y = lax.dot_general(h, w2_ref[0], (((1,), (1,)), ((), ())),
                            preferred_element_type=jnp.float32)

        @pl.when(f == 0)
        def _init():
            ys_ref[...] = y

        @pl.when(f != 0)
        def _acc():
            ys_ref[...] += y


def _sc_scatter(pos_hbm, w_hbm, rt_hbm, rw_hbm,
                zi_v, zf_v, ts_v, ws_v, idx_v, sem):
    cid = lax.axis_index("c")
    sid = lax.axis_index("s")

    @pl.when(cid == 0)
    def _():
        def zfill(i, c):
            zi_v[pl.ds(i * 16, 16)] = jnp.zeros((16,), jnp.int32)
            zf_v[pl.ds(i * 16, 16)] = jnp.zeros((16,), jnp.float32)
            return c

        lax.fori_loop(0, ZPT // 16, zfill, 0)
        pltpu.sync_copy(zi_v, rt_hbm.at[pl.ds(sid * ZPT, ZPT)])
        pltpu.sync_copy(zf_v, rw_hbm.at[pl.ds(sid * ZPT, ZPT)])
        plsc.subcore_barrier()

        tbase = sid * APT - S * (sid // (NT // TOPK))

        def build_ts(i, c):
            ts_v[pl.ds(i * 16, 16)] = (
                lax.broadcasted_iota(jnp.int32, (16,), 0) + (tbase + i * 16))
            return c

        lax.fori_loop(0, APT // 16, build_ts, 0)

        def scat(j, c):
            pltpu.sync_copy(pos_hbm.at[0, pl.ds(sid * APT + j * SCH, SCH)],
                            idx_v)
            pltpu.async_copy(ts_v.at[pl.ds(j * SCH, SCH)],
                             rt_hbm.at[idx_v], sem).wait()
            pltpu.sync_copy(w_hbm.at[0, pl.ds(sid * APT + j * SCH, SCH)], ws_v)
            pltpu.async_copy(ws_v, rw_hbm.at[idx_v], sem).wait()
            return c

        lax.fori_loop(0, APT // SCH, scat, 0)

    @pl.when(cid != 0)
    def _():
        plsc.subcore_barrier()


def _sc_gather(x_hbm, rt_hbm, xs_hbm, gidx_v, rows_v, sem):
    wid = lax.axis_index("s") * 2 + lax.axis_index("c")
    base = wid * (RP // NW)

    def body(i, c):
        b = base + i * GCH
        pltpu.sync_copy(rt_hbm.at[pl.ds(b, GCH)], gidx_v)
        pltpu.async_copy(x_hbm.at[gidx_v], rows_v, sem).wait()
        pltpu.sync_copy(rows_v, xs_hbm.at[pl.ds(b, GCH)])
        return c

    lax.fori_loop(0, (RP // NW) // GCH, body, 0)


def _sc_combine(ys_hbm, pos_hbm, out_hbm, i0_v, i1_v, a_v, b_v, sem):
    cid = lax.axis_index("c")
    sid = lax.axis_index("s")
    wid = sid * 2 + cid
    toks_per_w = S // NW
    base = wid * toks_per_w
    pltpu.sync_copy(pos_hbm.at[0, pl.ds(base, toks_per_w)], i0_v)
    pltpu.sync_copy(pos_hbm.at[0, pl.ds(S + base, toks_per_w)], i1_v)

    def body(i, c):
        coff = i * CCH
        pltpu.async_copy(ys_hbm.at[i0_v.at[pl.ds(coff, CCH)]], a_v, sem).wait()
        pltpu.async_copy(ys_hbm.at[i1_v.at[pl.ds(coff, CCH)]], b_v, sem).wait()

        def row(r, c2):
            for j in range(H // 16):
                sl = pl.ds(j * 16, 16)
                a_v[r, sl] = a_v[r, sl] + b_v[r, sl]
            return c2

        lax.fori_loop(0, CCH, row, 0)
        pltpu.sync_copy(a_v, out_hbm.at[pl.ds(base + coff, CCH)])
        return c

    lax.fori_loop(0, toks_per_w // CCH, body, 0)


def kernel(hidden_states, gate_weight, w1, w3, w2):
    B = hidden_states.shape[0]
    F = w1.shape[1]
    x = hidden_states.reshape(S, H)

    logits, plan_pos, plan_w, meta = pl.pallas_call(
        _router_body,
        out_shape=[
            jax.ShapeDtypeStruct((S, NE), jnp.float32),
            jax.ShapeDtypeStruct((NE, A), jnp.int32),
            jax.ShapeDtypeStruct((NE, A), jnp.float32),
            jax.ShapeDtypeStruct((1, NB + NE), jnp.int32),
        ],
    )(x, gate_weight)

    rt, rw = functools.partial(
        pl.kernel,
        mesh=plsc.VectorSubcoreMesh(core_axis_name="c", subcore_axis_name="s"),
        out_type=[
            jax.ShapeDtypeStruct((RP,), jnp.int32),
            jax.ShapeDtypeStruct((RP,), jnp.float32),
        ],
        scratch_types=[
            pltpu.VMEM((ZPT,), jnp.int32),
            pltpu.VMEM((ZPT,), jnp.float32),
            pltpu.VMEM((APT,), jnp.int32),
            pltpu.VMEM((SCH,), jnp.float32),
            pltpu.VMEM((SCH,), jnp.int32),
            pltpu.SemaphoreType.DMA,
        ],
    )(_sc_scatter)(plan_pos, plan_w)

    xs = functools.partial(
        pl.kernel,
        mesh=plsc.VectorSubcoreMesh(core_axis_name="c", subcore_axis_name="s"),
        out_type=jax.ShapeDtypeStruct((RP, H), jnp.float32),
        scratch_types=[
            pltpu.VMEM((GCH,), jnp.int32),
            pltpu.VMEM((GCH, H), jnp.float32),
            pltpu.SemaphoreType.DMA,
        ],
    )(_sc_gather)(x, rt)

    def _w13_map(b, f, m):
        dead = b >= m[0, NB]
        return (m[0, b], jnp.where(dead, F // FB - 1, f), 0)

    def _w2_map(b, f, m):
        dead = b >= m[0, NB]
        return (m[0, b], 0, jnp.where(dead, F // FB - 1, f))

    ys = pl.pallas_call(
        _gemm_body,
        grid_spec=pltpu.PrefetchScalarGridSpec(
            num_scalar_prefetch=1,
            grid=(NB, F // FB),
            in_specs=[
                pl.BlockSpec((BLK, H), lambda b, f, m: (b, 0)),
                pl.BlockSpec((1, BLK, 1), lambda b, f, m: (b, 0, 0)),
                pl.BlockSpec((1, FB, H), _w13_map),
                pl.BlockSpec((1, FB, H), _w13_map),
                pl.BlockSpec((1, H, FB), _w2_map),
            ],
            out_specs=pl.BlockSpec((BLK, H), lambda b, f, m: (b, 0)),
        ),
        out_shape=jax.ShapeDtypeStruct((RP, H), jnp.float32),
        compiler_params=pltpu.CompilerParams(
            dimension_semantics=("arbitrary", "arbitrary"),
        ),
    )(meta, xs, rw.reshape(NB, BLK, 1), w1, w3, w2)

    out = functools.partial(
        pl.kernel,
        mesh=plsc.VectorSubcoreMesh(core_axis_name="c", subcore_axis_name="s"),
        out_type=jax.ShapeDtypeStruct((S, H), jnp.float32),
        scratch_types=[
            pltpu.VMEM((S // NW,), jnp.int32),
            pltpu.VMEM((S // NW,), jnp.int32),
            pltpu.VMEM((CCH, H), jnp.float32),
            pltpu.VMEM((CCH, H), jnp.float32),
            pltpu.SemaphoreType.DMA,
        ],
    )(_sc_combine)(ys, plan_pos)

    return (out.reshape(B, S, H), logits)
